# baseline (speedup 1.0000x reference)
import functools

import jax
import jax.numpy as jnp
from jax import lax
from jax.experimental import pallas as pl
from jax.experimental.pallas import tpu as pltpu
from jax.experimental.pallas import tpu_sc as plsc

D = 64
B = 4096
NC, NS, L = 2, 16, 16
NW = NC * NS
RPW = B // NW
MAX_NORM = 1.0
EPS = 1e-7


def _permute(x, idx):
    dnums = lax.GatherDimensionNumbers(
        offset_dims=(), collapsed_slice_dims=(0,), start_index_map=(0,))
    return lax.gather(x, idx[:, None], dnums, slice_sizes=(1,),
                      mode=lax.GatherScatterMode.PROMISE_IN_BOUNDS)


def _rsqrt(x):
    i = plsc.bitcast(x, jnp.int32)
    i = jnp.int32(0x5F3759DF) - lax.shift_right_logical(i, 1)
    y = plsc.bitcast(i, jnp.float32)
    for _ in range(3):
        y = y * (1.5 - 0.5 * x * y * y)
    return y


def _scale_rows(idx_v, rows2, out_v):
    lane = lax.iota(jnp.int32, L)

    def group(g, carry):
        rids = g * L + lane
        offs = (idx_v[pl.ds(g * L, L)] & 1) * D

        def sumsq(d, acc):
            v = plsc.load_gather(rows2, [rids, offs + d])
            return acc + v * v

        acc = lax.fori_loop(0, D, sumsq, jnp.zeros((L,), jnp.float32))
        norm = acc * _rsqrt(acc)
        scale = jnp.where(acc > MAX_NORM * MAX_NORM,
                          MAX_NORM / (norm + EPS),
                          jnp.full((L,), 1.0, dtype=jnp.float32))
        flat0 = rids * D

        def emit(d, carry2):
            v = plsc.load_gather(rows2, [rids, offs + d])
            p = flat0 + d
            plsc.store_scatter(out_v, [lax.shift_right_logical(p, 7), p & 127],
                               v * scale)
            return carry2

        lax.fori_loop(0, D, emit, 0)
        return carry

    lax.fori_loop(0, RPW // L, group, 0)


@functools.partial(
    pl.kernel,
    out_type=(
        jax.ShapeDtypeStruct((B // 2, 2 * D), jnp.float32),
        jax.ShapeDtypeStruct((B // 2, 2 * D), jnp.float32),
        jax.ShapeDtypeStruct((B // 2, 2 * D), jnp.float32),
    ),
    mesh=plsc.VectorSubcoreMesh(core_axis_name="c", subcore_axis_name="s"),
    compiler_params=pltpu.CompilerParams(needs_layout_passes=False),
    scratch_types=[
        pltpu.VMEM((RPW,), jnp.int32),
        pltpu.VMEM((RPW,), jnp.int32),
        pltpu.VMEM((RPW,), jnp.int32),
        pltpu.VMEM((RPW,), jnp.int32),
        pltpu.VMEM((RPW,), jnp.int32),
        pltpu.VMEM((RPW,), jnp.int32),
        pltpu.VMEM((RPW, 2 * D), jnp.float32),
        pltpu.VMEM((RPW, 2 * D), jnp.float32),
        pltpu.VMEM((RPW, 2 * D), jnp.float32),
        pltpu.VMEM((RPW // 2, 2 * D), jnp.float32),
        pltpu.VMEM((RPW // 2, 2 * D), jnp.float32),
        pltpu.VMEM((RPW // 2, 2 * D), jnp.float32),
        pltpu.SemaphoreType.DMA,
        pltpu.SemaphoreType.DMA,
        pltpu.SemaphoreType.DMA,
        pltpu.SemaphoreType.DMA,
    ],
)
def _sc_lookup(ig, ia, ie, wg, wa, we, og, oa, oe,
               xg, xa, xe, hg, ha, he, rg, ra, re, ug, ua, ue,
               sg, sa, se, so):
    wid = lax.axis_index("s") * NC + lax.axis_index("c")
    base = wid * RPW
    copies = []
    for idx_hbm, idx_v, half_v, table, rows_v, sem in (
            (ig, xg, hg, wg, rg, sg),
            (ia, xa, ha, wa, ra, sa),
            (ie, xe, he, we, re, se)):
        pltpu.sync_copy(idx_hbm.at[pl.ds(base, RPW)], idx_v)
        for k in range(RPW // L):
            half_v[pl.ds(k * L, L)] = lax.shift_right_logical(
                idx_v[pl.ds(k * L, L)], 1)
        copies.append(pltpu.async_copy(table.at[half_v], rows_v, sem))
    out_copies = []
    for idx_v, rows_v, out_v, out_hbm, cp in (
            (xg, rg, ug, og, copies[0]),
            (xa, ra, ua, oa, copies[1]),
            (xe, re, ue, oe, copies[2])):
        cp.wait()
        _scale_rows(idx_v, rows_v, out_v)
        out_copies.append(pltpu.async_copy(
            out_v, out_hbm.at[pl.ds(wid * (RPW // 2), RPW // 2)], so))
    for cp in out_copies:
        cp.wait()


def kernel(latent_idx_geo, latent_idx_app, latent_idx_exp, W_geo, W_app, W_exp):
    og, oa, oe = _sc_lookup(latent_idx_geo.astype(jnp.int32),
                            latent_idx_app.astype(jnp.int32),
                            latent_idx_exp.astype(jnp.int32),
                            W_geo.reshape(-1, 2 * D),
                            W_app.reshape(-1, 2 * D),
                            W_exp.reshape(-1, 2 * D))
    return (og.reshape(B, D), oa.reshape(B, D), oe.reshape(B, D))


# speedup vs baseline: 1.6441x; 1.6441x over previous
import functools

import jax
import jax.numpy as jnp
from jax import lax
from jax.experimental import pallas as pl
from jax.experimental.pallas import tpu as pltpu
from jax.experimental.pallas import tpu_sc as plsc

D = 64
B = 4096
NC, NS, L = 2, 16, 16
NW = NC * NS
RPW = B // NW
MAX_NORM = 1.0
EPS = 1e-7


def _permute(x, idx):
    dnums = lax.GatherDimensionNumbers(
        offset_dims=(), collapsed_slice_dims=(0,), start_index_map=(0,))
    return lax.gather(x, idx[:, None], dnums, slice_sizes=(1,),
                      mode=lax.GatherScatterMode.PROMISE_IN_BOUNDS)


def _rsqrt(x):
    i = plsc.bitcast(x, jnp.int32)
    i = jnp.int32(0x5F3759DF) - lax.shift_right_logical(i, 1)
    y = plsc.bitcast(i, jnp.float32)
    for _ in range(3):
        y = y * (1.5 - 0.5 * x * y * y)
    return y


def _scale_rows(rows):
    def body(r, carry):
        v0 = rows[r, pl.ds(0 * L, L)]
        v1 = rows[r, pl.ds(1 * L, L)]
        v2 = rows[r, pl.ds(2 * L, L)]
        v3 = rows[r, pl.ds(3 * L, L)]
        acc = v0 * v0 + v1 * v1 + v2 * v2 + v3 * v3
        lane = lax.iota(jnp.int32, L)
        for k in (1, 2, 4, 8):
            acc = acc + _permute(acc, lane ^ k)
        norm = acc * _rsqrt(acc)
        scale = jnp.where(acc > MAX_NORM * MAX_NORM,
                          MAX_NORM / (norm + EPS),
                          jnp.full((L,), 1.0, dtype=jnp.float32))
        rows[r, pl.ds(0 * L, L)] = v0 * scale
        rows[r, pl.ds(1 * L, L)] = v1 * scale
        rows[r, pl.ds(2 * L, L)] = v2 * scale
        rows[r, pl.ds(3 * L, L)] = v3 * scale
        return carry

    lax.fori_loop(0, RPW, body, 0)


@functools.partial(
    pl.kernel,
    out_type=(
        jax.ShapeDtypeStruct((B, D), jnp.float32),
        jax.ShapeDtypeStruct((B, D), jnp.float32),
        jax.ShapeDtypeStruct((B, D), jnp.float32),
    ),
    mesh=plsc.VectorSubcoreMesh(core_axis_name="c", subcore_axis_name="s"),
    compiler_params=pltpu.CompilerParams(needs_layout_passes=False),
    scratch_types=[
        pltpu.VMEM((RPW,), jnp.int32),
        pltpu.VMEM((RPW,), jnp.int32),
        pltpu.VMEM((RPW,), jnp.int32),
        pltpu.VMEM((RPW, D), jnp.float32),
        pltpu.VMEM((RPW, D), jnp.float32),
        pltpu.VMEM((RPW, D), jnp.float32),
        pltpu.SemaphoreType.DMA,
        pltpu.SemaphoreType.DMA,
        pltpu.SemaphoreType.DMA,
    ],
)
def _sc_lookup(ig, ia, ie, wg, wa, we, og, oa, oe,
               xg, xa, xe, rg, ra, re, sg, sa, se):
    wid = lax.axis_index("s") * NC + lax.axis_index("c")
    base = wid * RPW
    for idx_hbm, idx_v, table, rows_v, sem in (
            (ig, xg, wg, rg, sg), (ia, xa, wa, ra, sa), (ie, xe, we, re, se)):
        pltpu.sync_copy(idx_hbm.at[pl.ds(base, RPW)], idx_v)
        for c in range(RPW // L):
            ids = idx_v[pl.ds(c * L, L)]
            for l in range(L):
                pltpu.async_copy(table.at[pl.ds(ids[l], 1), :],
                                 rows_v.at[pl.ds(c * L + l, 1), :], sem)
    for idx_hbm, rows_v, table, out_hbm, sem in (
            (ig, rg, wg, og, sg), (ia, ra, wa, oa, sa), (ie, re, we, oe, se)):
        pltpu.make_async_copy(table.at[pl.ds(0, RPW), :], rows_v, sem).wait()
        _scale_rows(rows_v)
        pltpu.sync_copy(rows_v, out_hbm.at[pl.ds(base, RPW)])


def kernel(latent_idx_geo, latent_idx_app, latent_idx_exp, W_geo, W_app, W_exp):
    return _sc_lookup(latent_idx_geo.astype(jnp.int32),
                      latent_idx_app.astype(jnp.int32),
                      latent_idx_exp.astype(jnp.int32),
                      W_geo, W_app, W_exp)
